# baseline (speedup 1.0000x reference)
import functools

import jax
import jax.numpy as jnp
from jax import lax
from jax.experimental import pallas as pl
from jax.experimental.pallas import tpu as pltpu
from jax.experimental.pallas import tpu_sc as plsc

_D = 32
_NW = 32
_NB = 512


@functools.lru_cache(maxsize=None)
def _build(n_tok: int, n_batch: int):
    jb_per_tok = n_batch // _NB
    n_items = n_tok * jb_per_tok
    per_w = n_items // _NW
    idx_per_w = per_w * _NB
    mesh = plsc.VectorSubcoreMesh(core_axis_name="c", subcore_axis_name="s")

    @functools.partial(
        pl.kernel,
        mesh=mesh,
        out_type=jax.ShapeDtypeStruct((n_tok, _D, n_batch), jnp.float32),
        compiler_params=pltpu.CompilerParams(
            use_tc_tiling_on_sc=False, needs_layout_passes=False),
        scratch_types=[
            pltpu.VMEM((idx_per_w,), jnp.int32),
            pltpu.VMEM((2, _NB, _D), jnp.float32),
            pltpu.VMEM((2, _D, _NB), jnp.float32),
            pltpu.SemaphoreType.DMA((2,)),
            pltpu.SemaphoreType.DMA((2,)),
        ],
    )
    def gather_kernel(idx_hbm, table_hbm, out_hbm, idx_v, rows_v, plane_v,
                      sem_g, sem_w):
        wid = lax.axis_index("s") * 2 + lax.axis_index("c")
        g0 = wid * per_w

        def fire_g(j, b):
            pltpu.async_copy(
                table_hbm.at[idx_v.at[pl.ds(j * _NB, _NB)]],
                rows_v.at[b], sem_g.at[b])

        def wait_g(j, b):
            pltpu.make_async_copy(
                table_hbm.at[idx_v.at[pl.ds(j * _NB, _NB)]],
                rows_v.at[b], sem_g.at[b]).wait()

        def out_slice(j):
            g = g0 + j
            t = g // jb_per_tok
            jb = g % jb_per_tok
            return out_hbm.at[t, :, pl.ds(pl.multiple_of(jb * _NB, 8), _NB)]

        def fire_w(j, b):
            pltpu.async_copy(plane_v.at[b], out_slice(j), sem_w.at[b])

        def wait_w(j, b):
            pltpu.make_async_copy(plane_v.at[b], out_slice(j),
                                  sem_w.at[b]).wait()

        iota16 = lax.iota(jnp.int32, 16)

        def transpose(b):
            def col_body(c, carry):
                row_ids = iota16 + c * 16
                for f in range(_D):
                    v = plsc.load_gather(
                        rows_v.at[b],
                        [row_ids, jnp.full((16,), f, jnp.int32)])
                    plane_v[b, f, pl.ds(c * 16, 16)] = v
                return carry
            lax.fori_loop(0, _NB // 16, col_body, 0)

        pltpu.sync_copy(
            idx_hbm.at[pl.ds(pl.multiple_of(g0 * _NB, 8), idx_per_w)], idx_v)

        fire_g(0, 0)
        fire_g(1, 1)
        wait_g(0, 0)
        transpose(0)
        fire_w(0, 0)
        fire_g(2, 0)
        wait_g(1, 1)
        transpose(1)
        fire_w(1, 1)
        fire_g(3, 1)

        def body(r, carry):
            for b in range(2):
                j = 2 * r + b
                wait_g(j, b)
                wait_w(j - 2, b)
                transpose(b)
                fire_w(j, b)
                fire_g(j + 2, b)
            return carry

        lax.fori_loop(1, per_w // 2 - 1, body, 0)

        for j in (per_w - 2, per_w - 1):
            b = j % 2
            wait_g(j, b)
            wait_w(j - 2, b)
            transpose(b)
            fire_w(j, b)
        for j in (per_w - 2, per_w - 1):
            wait_w(j, j % 2)

    return gather_kernel


def kernel(input_tensor, embeddings_tensor):
    nb, nt = input_tensor.shape
    idx_flat = input_tensor.T.reshape(-1)
    out_t = _build(nt, nb)(idx_flat, embeddings_tensor)
    return jnp.transpose(out_t, (2, 0, 1))


# speedup vs baseline: 1.1770x; 1.1770x over previous
import functools

import jax
import jax.numpy as jnp
from jax import lax
from jax.experimental import pallas as pl
from jax.experimental.pallas import tpu as pltpu
from jax.experimental.pallas import tpu_sc as plsc

_D = 32
_NW = 32
_NB = 512


@functools.lru_cache(maxsize=None)
def _build(n_tok: int, n_batch: int):
    jb_per_tok = n_batch // _NB
    n_items = n_tok * jb_per_tok
    per_w = n_items // _NW
    idx_per_w = per_w * _NB
    mesh = plsc.VectorSubcoreMesh(core_axis_name="c", subcore_axis_name="s")

    @functools.partial(
        pl.kernel,
        mesh=mesh,
        out_type=jax.ShapeDtypeStruct((n_tok, _D, n_batch), jnp.float32),
        compiler_params=pltpu.CompilerParams(
            use_tc_tiling_on_sc=False, needs_layout_passes=False),
        scratch_types=[
            pltpu.VMEM((idx_per_w,), jnp.int32),
            pltpu.VMEM((2, _NB, _D), jnp.float32),
            pltpu.VMEM((2, _D, _NB), jnp.float32),
            pltpu.SemaphoreType.DMA((2,)),
            pltpu.SemaphoreType.DMA((2,)),
        ],
    )
    def gather_kernel(idx_hbm, table_hbm, out_hbm, idx_v, rows_v, plane_v,
                      sem_g, sem_w):
        wid = lax.axis_index("s") * 2 + lax.axis_index("c")
        g0 = wid * per_w

        def fire_g(j, b):
            pltpu.async_copy(
                table_hbm.at[idx_v.at[pl.ds(j * _NB, _NB)]],
                rows_v.at[b], sem_g.at[b])

        def wait_g(j, b):
            pltpu.make_async_copy(
                table_hbm.at[idx_v.at[pl.ds(j * _NB, _NB)]],
                rows_v.at[b], sem_g.at[b]).wait()

        def out_slice(j):
            g = g0 + j
            t = g // jb_per_tok
            jb = g % jb_per_tok
            return out_hbm.at[t, :, pl.ds(pl.multiple_of(jb * _NB, 8), _NB)]

        def fire_w(j, b):
            pltpu.async_copy(plane_v.at[b], out_slice(j), sem_w.at[b])

        def wait_w(j, b):
            pltpu.make_async_copy(plane_v.at[b], out_slice(j),
                                  sem_w.at[b]).wait()

        iota16 = lax.iota(jnp.int32, 16)

        def transpose(b):
            @plsc.parallel_loop(0, _NB // 16, unroll=4)
            def col_body(c):
                row_ids = iota16 + c * 16
                for f in range(_D):
                    v = plsc.load_gather(
                        rows_v.at[b],
                        [row_ids, jnp.full((16,), f, jnp.int32)])
                    plane_v[b, f, pl.ds(c * 16, 16)] = v

        pltpu.sync_copy(
            idx_hbm.at[pl.ds(pl.multiple_of(g0 * _NB, 8), idx_per_w)], idx_v)

        fire_g(0, 0)
        fire_g(1, 1)
        wait_g(0, 0)
        transpose(0)
        fire_w(0, 0)
        fire_g(2, 0)
        wait_g(1, 1)
        transpose(1)
        fire_w(1, 1)
        fire_g(3, 1)

        def body(r, carry):
            for b in range(2):
                j = 2 * r + b
                wait_g(j, b)
                wait_w(j - 2, b)
                transpose(b)
                fire_w(j, b)
                fire_g(j + 2, b)
            return carry

        lax.fori_loop(1, per_w // 2 - 1, body, 0)

        for j in (per_w - 2, per_w - 1):
            b = j % 2
            wait_g(j, b)
            wait_w(j - 2, b)
            transpose(b)
            fire_w(j, b)
        for j in (per_w - 2, per_w - 1):
            wait_w(j, j % 2)

    return gather_kernel


def kernel(input_tensor, embeddings_tensor):
    nb, nt = input_tensor.shape
    idx_flat = input_tensor.T.reshape(-1)
    out_t = _build(nt, nb)(idx_flat, embeddings_tensor)
    return jnp.transpose(out_t, (2, 0, 1))
